# baseline (speedup 1.0000x reference)
import functools

import jax
import jax.numpy as jnp
from jax import lax
from jax.experimental import pallas as pl
from jax.experimental.pallas import tpu as pltpu
from jax.experimental.pallas import tpu_sc as plsc

_F32 = jnp.float32
_U = 8192
_CH = 128
_RB2 = 2048
_RB4 = 1024


def _s1_body(ce_ref, e_ref, wct_ref, wr_ref, wrat_ref,
             cemb_ref, trow_ref, acc_ref):
    k = pl.program_id(0)

    @pl.when(k == 0)
    def _init():
        acc_ref[...] = jnp.zeros_like(acc_ref)

    acc_ref[...] += jnp.dot(ce_ref[...], e_ref[...], preferred_element_type=_F32)

    @pl.when(k == pl.num_programs(0) - 1)
    def _fin():
        c_emb = jnp.dot(acc_ref[...], wct_ref[...], preferred_element_type=_F32)
        cemb_ref[...] = c_emb
        u_row = jnp.dot(wrat_ref[...], wr_ref[...], preferred_element_type=_F32)
        trow_ref[...] = jax.lax.dot_general(
            u_row, c_emb, (((1,), (1,)), ((), ())), preferred_element_type=_F32)


def _stage1(CE_adj, E, W_C_T, W_R, w_rating):
    C, N = CE_adj.shape
    D = E.shape[1]
    KB = 2048
    return pl.pallas_call(
        _s1_body,
        grid=(N // KB,),
        in_specs=[
            pl.BlockSpec((C, KB), lambda k: (0, k)),
            pl.BlockSpec((KB, D), lambda k: (k, 0)),
            pl.BlockSpec((D, D), lambda k: (0, 0)),
            pl.BlockSpec((D, D), lambda k: (0, 0)),
            pl.BlockSpec((1, D), lambda k: (0, 0)),
        ],
        out_specs=[
            pl.BlockSpec((C, D), lambda k: (0, 0)),
            pl.BlockSpec((1, C), lambda k: (0, 0)),
        ],
        out_shape=[
            jax.ShapeDtypeStruct((C, D), _F32),
            jax.ShapeDtypeStruct((1, C), _F32),
        ],
        scratch_shapes=[pltpu.VMEM((C, D), _F32)],
        compiler_params=pltpu.CompilerParams(dimension_semantics=("arbitrary",)),
    )(CE_adj, E, W_C_T, W_R, w_rating)


def _s2_body(rc_ref, trow_ref, out_ref):
    v = jax.lax.dot_general(
        trow_ref[...], rc_ref[...], (((1,), (1,)), ((), ())),
        preferred_element_type=_F32)
    out_ref[pl.ds(pl.program_id(0) % 8, 1), :] = v


def _stage2(RC_adj, t_row, base, nblk):
    R, C = RC_adj.shape
    return pl.pallas_call(
        _s2_body,
        grid=(nblk,),
        in_specs=[
            pl.BlockSpec((_RB2, C), lambda i: (base + i, 0)),
            pl.BlockSpec((1, C), lambda i: (0, 0)),
        ],
        out_specs=pl.BlockSpec((8, _RB2), lambda i: (i // 8, 0)),
        out_shape=jax.ShapeDtypeStruct((nblk, _RB2), _F32),
        compiler_params=pltpu.CompilerParams(dimension_semantics=("arbitrary",)),
    )(RC_adj, t_row)


def _make_sc_body(ept):
    nchunk = ept // _CH
    npair = nchunk // 2

    def _sc_body(ecat, dst, src3, rat, out, dst_v, src_v, rat_v, idx_a, idx_b,
                 sidx_a, sidx_b, rows_a, rows_b, acc, sem_a, sem_b):
        c = lax.axis_index("c")
        s = lax.axis_index("s")

        pltpu.sync_copy(dst.at[pl.ds(s * ept, ept)], dst_v)
        pltpu.sync_copy(src3.at[s], src_v)
        pltpu.sync_copy(rat.at[s], rat_v)

        @plsc.parallel_loop(0, _CH, unroll=4)
        def _zrow(r):
            for j in range(8):
                rows_a[r, pl.ds(j * 16, 16)] = jnp.zeros((16,), _F32)
        for q in range(4):
            pltpu.sync_copy(rows_a, acc.at[pl.ds(s * 512 + q * _CH, _CH)])
        plsc.subcore_barrier()


        def _fill_idx(ci, ibuf, sibuf):
            for j in range(8):
                sl = pl.ds(j * 16, 16)
                ibuf[sl] = dst_v[pl.ds(ci * _CH + j * 16, 16)] * 2 + c
                sibuf[sl] = src_v[ci, sl]

        def _fetch(ibuf, buf, sem):
            pltpu.async_copy(ecat.at[ibuf], buf, sem)

        def _fetch_wait(ibuf, buf, sem):
            pltpu.make_async_copy(ecat.at[ibuf], buf, sem).wait()

        def _scale_scatter(buf, sibuf, ci):
            @plsc.parallel_loop(0, _CH // 16, unroll=2)
            def _grp(g):
                vec = rat_v[pl.ds(ci * _CH + g * 16, 16)]
                for m in range(16):
                    rb = jnp.broadcast_to(vec[m], (16,))
                    r = g * 16 + m
                    for j in range(8):
                        sl = pl.ds(j * 16, 16)
                        buf[r, sl] = buf[r, sl] * rb

            pltpu.sync_copy(buf, acc.at[sibuf], add=True)

        _fill_idx(0, idx_a, sidx_a)
        _fetch(idx_a, rows_a, sem_a)

        def _pair(p, carry):
            i0 = 2 * p
            i1 = i0 + 1
            _fill_idx(i1, idx_b, sidx_b)
            _fetch(idx_b, rows_b, sem_b)
            _fetch_wait(idx_a, rows_a, sem_a)
            _scale_scatter(rows_a, sidx_a, i0)

            @pl.when(p < npair - 1)
            def _next():
                _fill_idx(i0 + 2, idx_a, sidx_a)
                _fetch(idx_a, rows_a, sem_a)

            _fetch_wait(idx_b, rows_b, sem_b)
            _scale_scatter(rows_b, sidx_b, i1)
            return carry

        lax.fori_loop(0, npair, _pair, 0)

        plsc.subcore_barrier()
        pltpu.sync_copy(acc.at[pl.ds(s * 512, 512)],
                        out.at[c, pl.ds(s * 512, 512)])

    return _sc_body


def _stage3(E_cat, dst, src3, rat):
    ept = dst.shape[0] // 16
    nchunk = ept // _CH
    mesh = plsc.VectorSubcoreMesh(core_axis_name="c", subcore_axis_name="s")
    k = functools.partial(
        pl.kernel,
        mesh=mesh,
        out_type=jax.ShapeDtypeStruct((2, _U, 128), _F32),
        scratch_types=[
            pltpu.VMEM((ept,), jnp.int32),
            pltpu.VMEM((nchunk, _CH), jnp.int32),
            pltpu.VMEM((ept,), _F32),
            pltpu.VMEM((_CH,), jnp.int32),
            pltpu.VMEM((_CH,), jnp.int32),
            pltpu.VMEM((_CH,), jnp.int32),
            pltpu.VMEM((_CH,), jnp.int32),
            pltpu.VMEM((_CH, 128), _F32),
            pltpu.VMEM((_CH, 128), _F32),
            pltpu.VMEM_SHARED((_U, 128), _F32),
            pltpu.SemaphoreType.DMA,
            pltpu.SemaphoreType.DMA,
        ],
    )(_make_sc_body(ept))
    return k(E_cat, dst, src3, rat)


def _s4_body(ec_ref, cemb_ref, out_ref):
    ec = jnp.dot(ec_ref[...], cemb_ref[...], preferred_element_type=_F32)
    out_ref[...] = jnp.maximum(ec, 0.01 * ec)


def _stage4(EC_adj, c_emb):
    N, C = EC_adj.shape
    D = c_emb.shape[1]
    return pl.pallas_call(
        _s4_body,
        grid=(N // _RB4,),
        in_specs=[
            pl.BlockSpec((_RB4, C), lambda i: (i, 0)),
            pl.BlockSpec((C, D), lambda i: (0, 0)),
        ],
        out_specs=pl.BlockSpec((_RB4, D), lambda i: (i, 0)),
        out_shape=jax.ShapeDtypeStruct((N, D), _F32),
        compiler_params=pltpu.CompilerParams(dimension_semantics=("arbitrary",)),
    )(EC_adj, c_emb)


def _s5_body(eo_ref, a0_ref, a1_ref, b0_ref, b1_ref, out_ref):
    uu = jnp.concatenate(
        [a0_ref[0] + b0_ref[0], a1_ref[0] + b1_ref[0]], axis=1)
    out_ref[...] = eo_ref[...] + jnp.maximum(uu, 0.01 * uu)


def _stage5(E_out, uu_a, uu_b):
    N, D = E_out.shape

    def _sp(half):
        return pl.BlockSpec((1, _RB4, 128), lambda i, h=half: (h, i, 0))

    return pl.pallas_call(
        _s5_body,
        grid=(_U // _RB4,),
        in_specs=[
            pl.BlockSpec((_RB4, D), lambda i: (i, 0)),
            _sp(0), _sp(1), _sp(0), _sp(1),
        ],
        out_specs=pl.BlockSpec((_RB4, D), lambda i: (i, 0)),
        out_shape=jax.ShapeDtypeStruct((N, D), _F32),
        input_output_aliases={0: 0},
        compiler_params=pltpu.CompilerParams(dimension_semantics=("arbitrary",)),
    )(E_out, uu_a, uu_a, uu_b, uu_b)


def kernel(E, CE_adj, EC_adj, RC_adj, UU_idx, W_C, W_R, w_rating):
    c_emb, t_row = _stage1(CE_adj, E, W_C.T, W_R, w_rating)
    e_rows = E.reshape(-1, 128)

    na = 32768
    src_all = UU_idx[0]
    dst_all = UU_idx[1]

    rat_a = _stage2(RC_adj, t_row, 0, na // _RB2)
    uu_a = _stage3(e_rows, dst_all[:na],
                   src_all[:na].reshape(16, -1, _CH), rat_a)
    rat_b = _stage2(RC_adj, t_row, na // _RB2, (RC_adj.shape[0] - na) // _RB2)
    uu_b = _stage3(e_rows, dst_all[na:],
                   src_all[na:].reshape(16, -1, _CH), rat_b)

    ec_msg = _stage4(EC_adj, c_emb)
    return _stage5(ec_msg, uu_a, uu_b)


# speedup vs baseline: 1.0614x; 1.0614x over previous
import functools

import jax
import jax.numpy as jnp
from jax import lax
from jax.experimental import pallas as pl
from jax.experimental.pallas import tpu as pltpu
from jax.experimental.pallas import tpu_sc as plsc

_F32 = jnp.float32
_U = 8192
_CH = 128
_RB2 = 2048
_RB4 = 1024


def _s1_body(ce_ref, e_ref, wct_ref, wr_ref, wrat_ref,
             cemb_ref, trow_ref, ecat_ref, acc_ref):
    k = pl.program_id(0)

    @pl.when(k == 0)
    def _init():
        acc_ref[...] = jnp.zeros_like(acc_ref)

    acc_ref[...] += jnp.dot(ce_ref[...], e_ref[...], preferred_element_type=_F32)

    @pl.when(k < 4)
    def _cat():
        ecat_ref[0] = e_ref[:, :128]
        ecat_ref[1] = e_ref[:, 128:]

    @pl.when(k == pl.num_programs(0) - 1)
    def _fin():
        c_emb = jnp.dot(acc_ref[...], wct_ref[...], preferred_element_type=_F32)
        cemb_ref[...] = c_emb
        u_row = jnp.dot(wrat_ref[...], wr_ref[...], preferred_element_type=_F32)
        trow_ref[...] = jax.lax.dot_general(
            u_row, c_emb, (((1,), (1,)), ((), ())), preferred_element_type=_F32)


def _stage1(CE_adj, E, W_C_T, W_R, w_rating):
    C, N = CE_adj.shape
    D = E.shape[1]
    KB = 2048
    return pl.pallas_call(
        _s1_body,
        grid=(N // KB,),
        in_specs=[
            pl.BlockSpec((C, KB), lambda k: (0, k)),
            pl.BlockSpec((KB, D), lambda k: (k, 0)),
            pl.BlockSpec((D, D), lambda k: (0, 0)),
            pl.BlockSpec((D, D), lambda k: (0, 0)),
            pl.BlockSpec((1, D), lambda k: (0, 0)),
        ],
        out_specs=[
            pl.BlockSpec((C, D), lambda k: (0, 0)),
            pl.BlockSpec((1, C), lambda k: (0, 0)),
            pl.BlockSpec((2, KB, 128), lambda k: (0, jnp.minimum(k, 3), 0)),
        ],
        out_shape=[
            jax.ShapeDtypeStruct((C, D), _F32),
            jax.ShapeDtypeStruct((1, C), _F32),
            jax.ShapeDtypeStruct((2, _U, 128), _F32),
        ],
        scratch_shapes=[pltpu.VMEM((C, D), _F32)],
        compiler_params=pltpu.CompilerParams(dimension_semantics=("arbitrary",)),
    )(CE_adj, E, W_C_T, W_R, w_rating)


def _s2_body(rc_ref, trow_ref, out_ref):
    v = jax.lax.dot_general(
        trow_ref[...], rc_ref[...], (((1,), (1,)), ((), ())),
        preferred_element_type=_F32)
    out_ref[pl.ds(pl.program_id(0) % 8, 1), :] = v


def _stage2(RC_adj, t_row, base, nblk):
    R, C = RC_adj.shape
    return pl.pallas_call(
        _s2_body,
        grid=(nblk,),
        in_specs=[
            pl.BlockSpec((_RB2, C), lambda i: (base + i, 0)),
            pl.BlockSpec((1, C), lambda i: (0, 0)),
        ],
        out_specs=pl.BlockSpec((8, _RB2), lambda i: (i // 8, 0)),
        out_shape=jax.ShapeDtypeStruct((nblk, _RB2), _F32),
        compiler_params=pltpu.CompilerParams(dimension_semantics=("arbitrary",)),
    )(RC_adj, t_row)


def _make_sc_body(ept):
    nchunk = ept // _CH
    npair = nchunk // 2

    def _sc_body(ecat, dst, src3, rat, out, dst_v, src_v, rat_v, idx_a, idx_b,
                 sidx_a, sidx_b, rows_a, rows_b, acc, sem_a, sem_b):
        c = lax.axis_index("c")
        s = lax.axis_index("s")

        pltpu.sync_copy(dst.at[pl.ds(s * ept, ept)], dst_v)
        pltpu.sync_copy(src3.at[s], src_v)
        pltpu.sync_copy(rat.at[s], rat_v)

        @plsc.parallel_loop(0, _CH, unroll=4)
        def _zrow(r):
            for j in range(8):
                rows_a[r, pl.ds(j * 16, 16)] = jnp.zeros((16,), _F32)
        for q in range(4):
            pltpu.sync_copy(rows_a, acc.at[pl.ds(s * 512 + q * _CH, _CH)])
        plsc.subcore_barrier()

        off = c * _U

        def _fill_idx(ci, ibuf, sibuf):
            for j in range(8):
                sl = pl.ds(j * 16, 16)
                ibuf[sl] = dst_v[pl.ds(ci * _CH + j * 16, 16)] + off
                sibuf[sl] = src_v[ci, sl]

        def _fetch(ibuf, buf, sem):
            pltpu.async_copy(ecat.at[ibuf], buf, sem)

        def _fetch_wait(ibuf, buf, sem):
            pltpu.make_async_copy(ecat.at[ibuf], buf, sem).wait()

        def _scale_scatter(buf, sibuf, ci):
            @plsc.parallel_loop(0, _CH // 16, unroll=2)
            def _grp(g):
                vec = rat_v[pl.ds(ci * _CH + g * 16, 16)]
                for m in range(16):
                    rb = jnp.broadcast_to(vec[m], (16,))
                    r = g * 16 + m
                    for j in range(8):
                        sl = pl.ds(j * 16, 16)
                        buf[r, sl] = buf[r, sl] * rb

            pltpu.sync_copy(buf, acc.at[sibuf], add=True)

        _fill_idx(0, idx_a, sidx_a)
        _fetch(idx_a, rows_a, sem_a)

        def _pair(p, carry):
            i0 = 2 * p
            i1 = i0 + 1
            _fill_idx(i1, idx_b, sidx_b)
            _fetch(idx_b, rows_b, sem_b)
            _fetch_wait(idx_a, rows_a, sem_a)
            _scale_scatter(rows_a, sidx_a, i0)

            @pl.when(p < npair - 1)
            def _next():
                _fill_idx(i0 + 2, idx_a, sidx_a)
                _fetch(idx_a, rows_a, sem_a)

            _fetch_wait(idx_b, rows_b, sem_b)
            _scale_scatter(rows_b, sidx_b, i1)
            return carry

        lax.fori_loop(0, npair, _pair, 0)

        plsc.subcore_barrier()
        pltpu.sync_copy(acc.at[pl.ds(s * 512, 512)],
                        out.at[c, pl.ds(s * 512, 512)])

    return _sc_body


def _stage3(E_cat, dst, src3, rat):
    ept = dst.shape[0] // 16
    nchunk = ept // _CH
    mesh = plsc.VectorSubcoreMesh(core_axis_name="c", subcore_axis_name="s")
    k = functools.partial(
        pl.kernel,
        mesh=mesh,
        out_type=jax.ShapeDtypeStruct((2, _U, 128), _F32),
        scratch_types=[
            pltpu.VMEM((ept,), jnp.int32),
            pltpu.VMEM((nchunk, _CH), jnp.int32),
            pltpu.VMEM((ept,), _F32),
            pltpu.VMEM((_CH,), jnp.int32),
            pltpu.VMEM((_CH,), jnp.int32),
            pltpu.VMEM((_CH,), jnp.int32),
            pltpu.VMEM((_CH,), jnp.int32),
            pltpu.VMEM((_CH, 128), _F32),
            pltpu.VMEM((_CH, 128), _F32),
            pltpu.VMEM_SHARED((_U, 128), _F32),
            pltpu.SemaphoreType.DMA,
            pltpu.SemaphoreType.DMA,
        ],
    )(_make_sc_body(ept))
    return k(E_cat, dst, src3, rat)


def _s4_body(ec_ref, cemb_ref, out_ref):
    ec = jnp.dot(ec_ref[...], cemb_ref[...], preferred_element_type=_F32)
    out_ref[...] = jnp.maximum(ec, 0.01 * ec)


def _stage4(EC_adj, c_emb):
    N, C = EC_adj.shape
    D = c_emb.shape[1]
    return pl.pallas_call(
        _s4_body,
        grid=(N // _RB4,),
        in_specs=[
            pl.BlockSpec((_RB4, C), lambda i: (i, 0)),
            pl.BlockSpec((C, D), lambda i: (0, 0)),
        ],
        out_specs=pl.BlockSpec((_RB4, D), lambda i: (i, 0)),
        out_shape=jax.ShapeDtypeStruct((N, D), _F32),
        compiler_params=pltpu.CompilerParams(dimension_semantics=("arbitrary",)),
    )(EC_adj, c_emb)


def _s5_body(eo_ref, a0_ref, a1_ref, b0_ref, b1_ref, out_ref):
    uu = jnp.concatenate(
        [a0_ref[0] + b0_ref[0], a1_ref[0] + b1_ref[0]], axis=1)
    out_ref[...] = eo_ref[...] + jnp.maximum(uu, 0.01 * uu)


def _stage5(E_out, uu_a, uu_b):
    N, D = E_out.shape

    def _sp(half):
        return pl.BlockSpec((1, _RB4, 128), lambda i, h=half: (h, i, 0))

    return pl.pallas_call(
        _s5_body,
        grid=(_U // _RB4,),
        in_specs=[
            pl.BlockSpec((_RB4, D), lambda i: (i, 0)),
            _sp(0), _sp(1), _sp(0), _sp(1),
        ],
        out_specs=pl.BlockSpec((_RB4, D), lambda i: (i, 0)),
        out_shape=jax.ShapeDtypeStruct((N, D), _F32),
        input_output_aliases={0: 0},
        compiler_params=pltpu.CompilerParams(dimension_semantics=("arbitrary",)),
    )(E_out, uu_a, uu_a, uu_b, uu_b)


def kernel(E, CE_adj, EC_adj, RC_adj, UU_idx, W_C, W_R, w_rating):
    c_emb, t_row, ecat = _stage1(CE_adj, E, W_C.T, W_R, w_rating)
    e_rows = ecat.reshape(2 * _U, 128)

    na = 32768
    src_all = UU_idx[0]
    dst_all = UU_idx[1]

    rat_a = _stage2(RC_adj, t_row, 0, na // _RB2)
    uu_a = _stage3(e_rows, dst_all[:na],
                   src_all[:na].reshape(16, -1, _CH), rat_a)
    rat_b = _stage2(RC_adj, t_row, na // _RB2, (RC_adj.shape[0] - na) // _RB2)
    uu_b = _stage3(e_rows, dst_all[na:],
                   src_all[na:].reshape(16, -1, _CH), rat_b)

    ec_msg = _stage4(EC_adj, c_emb)
    return _stage5(ec_msg, uu_a, uu_b)
